# Initial kernel scaffold; baseline (speedup 1.0000x reference)
import jax
import jax.numpy as jnp
from jax.experimental import pallas as pl


def kernel(x, edge_index, t1, W1l, W1r, ln1_w, ln1_b, t2, W2l, W2r, ln2_w, ln2_b, mem_k, mem_conv_w, mem_lin_w, fx_w, fx_b):
    raise NotImplementedError("write your pallas kernel here")


# speedup vs baseline: 1.9068x; 1.9068x over previous
import functools

import jax
import jax.numpy as jnp
from jax.experimental import pallas as pl
from jax.experimental.pallas import tpu as pltpu

N_NODES = 10000
N_EDGES = 320000
ROW_BLK = 2000
N_GRID = N_NODES // ROW_BLK


def _colmax_body(x_ref, t_ref, m_ref):
    i = pl.program_id(0)
    mx = jnp.max(x_ref[...] * t_ref[...], axis=0, keepdims=True)

    @pl.when(i == 0)
    def _():
        m_ref[...] = mx

    @pl.when(i > 0)
    def _():
        m_ref[...] = jnp.maximum(m_ref[...], mx)


def _colmax(x, t):
    n, d = x.shape
    return pl.pallas_call(
        _colmax_body,
        grid=(N_GRID,),
        in_specs=[
            pl.BlockSpec((ROW_BLK, d), lambda i: (i, 0)),
            pl.BlockSpec((1, d), lambda i: (0, 0)),
        ],
        out_specs=pl.BlockSpec((1, d), lambda i: (0, 0)),
        out_shape=jax.ShapeDtypeStruct((1, d), jnp.float32),
    )(x, t)


def _sage_dense_body(num_ref, den_ref, x_ref, wl_ref, wr_ref, y_ref, s1_ref, s2_ref):
    i = pl.program_id(0)
    aggr = num_ref[...] / (den_ref[...] + 1e-16)
    y = jnp.dot(aggr, wl_ref[...], preferred_element_type=jnp.float32)
    y += jnp.dot(x_ref[...], wr_ref[...], preferred_element_type=jnp.float32)
    y_ref[...] = y
    s1 = jnp.sum(y).reshape(1, 1)
    s2 = jnp.sum(y * y).reshape(1, 1)

    @pl.when(i == 0)
    def _():
        s1_ref[...] = s1
        s2_ref[...] = s2

    @pl.when(i > 0)
    def _():
        s1_ref[...] += s1
        s2_ref[...] += s2


def _sage_dense(num, den, x, wlt, wrt):
    n, d = x.shape
    h = wlt.shape[1]
    return pl.pallas_call(
        _sage_dense_body,
        grid=(N_GRID,),
        in_specs=[
            pl.BlockSpec((ROW_BLK, d), lambda i: (i, 0)),
            pl.BlockSpec((ROW_BLK, d), lambda i: (i, 0)),
            pl.BlockSpec((ROW_BLK, d), lambda i: (i, 0)),
            pl.BlockSpec((d, h), lambda i: (0, 0)),
            pl.BlockSpec((d, h), lambda i: (0, 0)),
        ],
        out_specs=[
            pl.BlockSpec((ROW_BLK, h), lambda i: (i, 0)),
            pl.BlockSpec((1, 1), lambda i: (0, 0)),
            pl.BlockSpec((1, 1), lambda i: (0, 0)),
        ],
        out_shape=[
            jax.ShapeDtypeStruct((n, h), jnp.float32),
            jax.ShapeDtypeStruct((1, 1), jnp.float32),
            jax.ShapeDtypeStruct((1, 1), jnp.float32),
        ],
    )(num, den, x, wlt, wrt)


def _norm_relu_body(n_elems, y_ref, s1_ref, s2_ref, w_ref, b_ref, t_ref, h_ref, m_ref):
    i = pl.program_id(0)
    mu = s1_ref[0, 0] / n_elems
    var = jnp.maximum(s2_ref[0, 0] / n_elems - mu * mu, 0.0)
    inv = 1.0 / (jnp.sqrt(var) + 1e-5)
    h = jnp.maximum((y_ref[...] - mu) * inv * w_ref[...] + b_ref[...], 0.0)
    h_ref[...] = h
    mx = jnp.max(h * t_ref[...], axis=0, keepdims=True)

    @pl.when(i == 0)
    def _():
        m_ref[...] = mx

    @pl.when(i > 0)
    def _():
        m_ref[...] = jnp.maximum(m_ref[...], mx)


def _norm_relu(y, s1, s2, w, b, t):
    n, h = y.shape
    return pl.pallas_call(
        functools.partial(_norm_relu_body, float(n * h)),
        grid=(N_GRID,),
        in_specs=[
            pl.BlockSpec((ROW_BLK, h), lambda i: (i, 0)),
            pl.BlockSpec((1, 1), lambda i: (0, 0)),
            pl.BlockSpec((1, 1), lambda i: (0, 0)),
            pl.BlockSpec((1, h), lambda i: (0, 0)),
            pl.BlockSpec((1, h), lambda i: (0, 0)),
            pl.BlockSpec((1, h), lambda i: (0, 0)),
        ],
        out_specs=[
            pl.BlockSpec((ROW_BLK, h), lambda i: (i, 0)),
            pl.BlockSpec((1, h), lambda i: (0, 0)),
        ],
        out_shape=[
            jax.ShapeDtypeStruct((n, h), jnp.float32),
            jax.ShapeDtypeStruct((1, h), jnp.float32),
        ],
    )(y, s1, s2, w, b, t)


def _final_body(n_elems, y_ref, s1_ref, s2_ref, w_ref, b_ref, mlw_ref, fxw_ref,
                fxb_ref, out_ref, acc_ref):
    i = pl.program_id(0)
    mu = s1_ref[0, 0] / n_elems
    var = jnp.maximum(s2_ref[0, 0] / n_elems - mu * mu, 0.0)
    inv = 1.0 / (jnp.sqrt(var) + 1e-5)
    h = jnp.maximum((y_ref[...] - mu) * inv * w_ref[...] + b_ref[...], 0.0)
    cs = jnp.sum(h, axis=0, keepdims=True)

    @pl.when(i == 0)
    def _():
        acc_ref[...] = cs

    @pl.when(i > 0)
    def _():
        acc_ref[...] += cs

    @pl.when(i == pl.num_programs(0) - 1)
    def _():
        pooled = jnp.dot(acc_ref[...], mlw_ref[...],
                         preferred_element_type=jnp.float32)
        out_ref[...] = jnp.dot(pooled, fxw_ref[...],
                               preferred_element_type=jnp.float32) + fxb_ref[...]


def _final(y, s1, s2, w, b, mlwt, fxwt, fxb):
    n, h = y.shape
    return pl.pallas_call(
        functools.partial(_final_body, float(n * h)),
        grid=(N_GRID,),
        in_specs=[
            pl.BlockSpec((ROW_BLK, h), lambda i: (i, 0)),
            pl.BlockSpec((1, 1), lambda i: (0, 0)),
            pl.BlockSpec((1, 1), lambda i: (0, 0)),
            pl.BlockSpec((1, h), lambda i: (0, 0)),
            pl.BlockSpec((1, h), lambda i: (0, 0)),
            pl.BlockSpec(mlwt.shape, lambda i: (0, 0)),
            pl.BlockSpec(fxwt.shape, lambda i: (0, 0)),
            pl.BlockSpec((1, fxwt.shape[1]), lambda i: (0, 0)),
        ],
        out_specs=pl.BlockSpec((1, fxwt.shape[1]), lambda i: (0, 0)),
        out_shape=jax.ShapeDtypeStruct((1, fxwt.shape[1]), jnp.float32),
        scratch_shapes=[pltpu.VMEM((1, h), jnp.float32)],
    )(y, s1, s2, w, b, mlwt, fxwt, fxb)


def _seg_sums(x, src, dst, t, m):
    xj = x[src]
    e = jnp.exp(xj * t - m)
    den = jax.ops.segment_sum(e, dst, num_segments=N_NODES)
    num = jax.ops.segment_sum(e * xj, dst, num_segments=N_NODES)
    return num, den


def kernel(x, edge_index, t1, W1l, W1r, ln1_w, ln1_b, t2, W2l, W2r, ln2_w,
           ln2_b, mem_k, mem_conv_w, mem_lin_w, fx_w, fx_b):
    src = edge_index[0]
    dst = edge_index[1]

    m1 = _colmax(x, t1)
    num1, den1 = _seg_sums(x, src, dst, t1, m1)
    y1, s1a, s1b = _sage_dense(num1, den1, x, W1l.T, W1r.T)
    h1, m2 = _norm_relu(y1, s1a, s1b, ln1_w.reshape(1, -1),
                        ln1_b.reshape(1, -1), t2)

    num2, den2 = _seg_sums(h1, src, dst, t2, m2)
    y2, s2a, s2b = _sage_dense(num2, den2, h1, W2l.T, W2r.T)

    return _final(y2, s2a, s2b, ln2_w.reshape(1, -1), ln2_b.reshape(1, -1),
                  mem_lin_w.T, fx_w.T, fx_b.reshape(1, -1))
